# Initial kernel scaffold; baseline (speedup 1.0000x reference)
import jax
import jax.numpy as jnp
from jax.experimental import pallas as pl


def kernel(feat0, feat1, feat2, feat3, feat4, labels, boxes):
    raise NotImplementedError("write your pallas kernel here")


# speedup vs baseline: 2.8230x; 2.8230x over previous
import functools

import numpy as np
import jax
import jax.numpy as jnp
from jax import lax
from jax.experimental import pallas as pl
from jax.experimental.pallas import tpu as pltpu
from jax.experimental.pallas import tpu_sc as plsc

_STRIDES = (8, 16, 32, 64, 128)
_BOUNDS = ((-1.0, 64.0), (64.0, 128.0), (128.0, 256.0),
           (256.0, 512.0), (512.0, 999999.0))
_IMG = 640
_B = 8
_NB = 50
_NBP = 64
_BIG = float(np.float32(99999999.0))

_LEVEL_N = tuple((_IMG // s) ** 2 for s in _STRIDES)
_NTOT = sum(_LEVEL_N)
_NTILES = 32
_LANES = 16
_NPAD = 8704
_CHUNK = _NPAD // _NTILES
_NVEC = _CHUNK // _LANES


def _build_locparams():
    rows = [[] for _ in range(6)]
    for s, (blo, bhi) in zip(_STRIDES, _BOUNDS):
        h = _IMG // s
        c = ((np.arange(h) + 0.5) * s).astype(np.float32)
        n = h * h
        rows[0].append(np.tile(c, h))
        rows[1].append(np.repeat(c, h))
        rows[2].append(np.full(n, blo, np.float32))
        rows[3].append(np.full(n, bhi, np.float32))
        rows[4].append(np.full(n, 1.5 * s, np.float32))
        rows[5].append(np.full(n, 1.0 / s, np.float32))
    fills = (-1e9, -1e9, 0.0, 1.0, 1.0, 1.0)
    out = np.stack([
        np.pad(np.concatenate(r), (0, _NPAD - _NTOT),
               constant_values=f).astype(np.float32)
        for r, f in zip(rows, fills)
    ])
    return out


_LOCP = _build_locparams()


def _sqrt_newton(x):
    yi = (plsc.bitcast(x, jnp.int32) >> 1) + jnp.int32(0x1FBD1DF5)
    y = plsc.bitcast(yi, jnp.float32)
    for _ in range(3):
        y = 0.5 * (y + x / y)
    return y


def _sc_body(boxp_hbm, lab_hbm, locp_hbm,
             cls_hbm, l_hbm, t_hbm, r_hbm, b_hbm, ctr_hbm,
             boxp_v, lab_v, locp_v, outf_v, outc_v):
    c = lax.axis_index("c")
    s = lax.axis_index("s")
    wid = s * 2 + c
    base = wid * _CHUNK
    for j in range(6):
        pltpu.sync_copy(locp_hbm.at[pl.ds(j * _NPAD + base, _CHUNK)],
                        locp_v.at[j])
    pltpu.sync_copy(boxp_hbm, boxp_v)
    pltpu.sync_copy(lab_hbm, lab_v)

    for b in range(_B):
        def v_body(v, _, b=b):
            off = v * _LANES
            x = locp_v[0, pl.ds(off, _LANES)]
            yy = locp_v[1, pl.ds(off, _LANES)]
            blo = locp_v[2, pl.ds(off, _LANES)]
            bhi = locp_v[3, pl.ds(off, _LANES)]
            rad = locp_v[4, pl.ds(off, _LANES)]
            inv = locp_v[5, pl.ds(off, _LANES)]

            def k_body(k, carry, b=b):
                best, bl, bt, br, bb, blab = carry
                kb = jnp.full((_LANES,), b * _NBP, jnp.int32) + k
                x1 = plsc.load_gather(boxp_v, [kb])
                y1 = plsc.load_gather(boxp_v, [kb + (_B * _NBP)])
                x2 = plsc.load_gather(boxp_v, [kb + (2 * _B * _NBP)])
                y2 = plsc.load_gather(boxp_v, [kb + (3 * _B * _NBP)])
                labb = plsc.load_gather(lab_v, [kb])
                l = x - x1
                t = yy - y1
                r = x2 - x
                d = y2 - yy
                mn = jnp.minimum(jnp.minimum(l, t), jnp.minimum(r, d))
                mx = jnp.maximum(jnp.maximum(l, t), jnp.maximum(r, d))
                cx = (x1 + x2) * 0.5
                cy = (y1 + y2) * 0.5
                ctrm = jnp.maximum(jnp.abs(x - cx), jnp.abs(yy - cy)) < rad
                pos = (mn > 0.0) & (mx > blo) & (mx <= bhi) & ctrm
                area = (l + r) * (t + d)
                am = jnp.where(pos, area, _BIG)
                take = am < best
                return (jnp.where(take, am, best),
                        jnp.where(take, l, bl),
                        jnp.where(take, t, bt),
                        jnp.where(take, r, br),
                        jnp.where(take, d, bb),
                        jnp.where(take, labb, blab))

            zf = jnp.zeros((_LANES,), jnp.float32)
            carry0 = (jnp.full((_LANES,), _BIG, jnp.float32),
                      zf, zf, zf, zf, jnp.zeros((_LANES,), jnp.int32))
            best, bl, bt, br, bb, blab = lax.fori_loop(0, _NB, k_body, carry0)

            neg = best >= _BIG
            lreg = bl * inv
            treg = bt * inv
            rreg = br * inv
            breg = bb * inv
            lrmin = jnp.minimum(lreg, rreg)
            lrmax = jnp.maximum(lreg, rreg)
            tbmin = jnp.minimum(treg, breg)
            tbmax = jnp.maximum(treg, breg)
            ratio = jnp.maximum(lrmin * tbmin / jnp.maximum(lrmax * tbmax, 1e-8),
                                1e-12)
            ct = _sqrt_newton(ratio)
            outc_v[b, pl.ds(off, _LANES)] = jnp.where(neg, 0, blab)
            outf_v[0, b, pl.ds(off, _LANES)] = jnp.where(neg, -1.0, lreg)
            outf_v[1, b, pl.ds(off, _LANES)] = jnp.where(neg, -1.0, treg)
            outf_v[2, b, pl.ds(off, _LANES)] = jnp.where(neg, -1.0, rreg)
            outf_v[3, b, pl.ds(off, _LANES)] = jnp.where(neg, -1.0, breg)
            outf_v[4, b, pl.ds(off, _LANES)] = jnp.where(neg, -1.0, ct)
            return 0

        lax.fori_loop(0, _NVEC, v_body, 0)

    fouts = (l_hbm, t_hbm, r_hbm, b_hbm, ctr_hbm)
    for b in range(_B):
        pltpu.sync_copy(outc_v.at[b],
                        cls_hbm.at[pl.ds(b * _NPAD + base, _CHUNK)])
        for j in range(5):
            pltpu.sync_copy(outf_v.at[j, b],
                            fouts[j].at[pl.ds(b * _NPAD + base, _CHUNK)])


_fcos_call = functools.partial(
    pl.kernel,
    out_type=(
        jax.ShapeDtypeStruct((_B * _NPAD,), jnp.int32),
        jax.ShapeDtypeStruct((_B * _NPAD,), jnp.float32),
        jax.ShapeDtypeStruct((_B * _NPAD,), jnp.float32),
        jax.ShapeDtypeStruct((_B * _NPAD,), jnp.float32),
        jax.ShapeDtypeStruct((_B * _NPAD,), jnp.float32),
        jax.ShapeDtypeStruct((_B * _NPAD,), jnp.float32),
    ),
    mesh=plsc.VectorSubcoreMesh(core_axis_name="c", subcore_axis_name="s",
                                num_cores=2, num_subcores=16),
    compiler_params=pltpu.CompilerParams(needs_layout_passes=False,
                                         use_tc_tiling_on_sc=False),
    scratch_types=[
        pltpu.VMEM((4 * _B * _NBP,), jnp.float32),
        pltpu.VMEM((_B * _NBP,), jnp.int32),
        pltpu.VMEM((6, _CHUNK), jnp.float32),
        pltpu.VMEM((5, _B, _CHUNK), jnp.float32),
        pltpu.VMEM((_B, _CHUNK), jnp.int32),
    ],
)(_sc_body)


def kernel(feat0, feat1, feat2, feat3, feat4, labels, boxes):
    bp = jnp.pad(jnp.moveaxis(boxes.astype(jnp.float32), -1, 0),
                 ((0, 0), (0, 0), (0, _NBP - _NB)))
    bp = bp.reshape(4 * _B * _NBP)
    lab = jnp.pad(labels.astype(jnp.int32),
                  ((0, 0), (0, _NBP - _NB))).reshape(_B * _NBP)
    locp = jnp.asarray(_LOCP.reshape(6 * _NPAD))
    outs = _fcos_call(bp, lab, locp)
    cls_f, l_f, t_f, r_f, b_f, ct_f = (o.reshape(_B, _NPAD) for o in outs)
    reg_f = jnp.stack([l_f, t_f, r_f, b_f], axis=-1)
    cls_l, reg_l, ctr_l = [], [], []
    o = 0
    for n in _LEVEL_N:
        cls_l.append(cls_f[:, o:o + n, None])
        reg_l.append(reg_f[:, o:o + n, :])
        ctr_l.append(ct_f[:, o:o + n, None])
        o += n
    return tuple(cls_l), tuple(reg_l), tuple(ctr_l)
